# baseline (speedup 1.0000x reference)
import functools

import numpy as np
import jax
import jax.numpy as jnp
from jax.experimental import pallas as pl
from jax.experimental.pallas import tpu as pltpu

_B, _N = 32, 512
_K = 128
_H = 64
_NPAIR = _K * (_K - 1) // 2
_UC = float(np.sqrt(_N * (_N + 1) / 12.0))


def _loss_kernel(wd_r, bo_r, sd_r, sdT_r, t_r, tT_r, s_r, sT_r,
                 w0_r, c_r, wh_r, bh_r, wo_r, out_r, acc_r):
    b = pl.program_id(0)

    @pl.when(b == 0)
    def _init():
        acc_r[0, 0] = 0.0
        acc_r[0, 1] = 0.0

    x_row = sd_r[0]
    x_col = sdT_r[0]

    ri = jax.lax.broadcasted_iota(jnp.int32, (_N, _N), 0)
    cj = jax.lax.broadcasted_iota(jnp.int32, (_N, _N), 1)
    eq = x_row == x_col
    m_i = ((x_row > x_col) | (eq & (cj < ri))).astype(jnp.int32)
    m_j = ((x_col > x_row) | (eq & (ri < cj))).astype(jnp.int32)
    rank_col = jnp.sum(m_i, axis=1, keepdims=True)
    rank_row = jnp.sum(m_j, axis=0, keepdims=True)

    kk_col = jax.lax.broadcasted_iota(jnp.int32, (_N, _K), 1)
    oh_ik = (rank_col == kk_col).astype(jnp.float32)
    kk_row = jax.lax.broadcasted_iota(jnp.int32, (_K, _N), 0)
    oh_kj = (kk_row == rank_row).astype(jnp.float32)

    dot = functools.partial(jnp.dot, preferred_element_type=jnp.float32)
    d_row = dot(x_row, oh_ik)
    t_row = dot(t_r[0], oh_ik)
    s_row = dot(s_r[0], oh_ik)
    d_col = dot(oh_kj, x_col)
    t_col = dot(oh_kj, tT_r[0])
    s_col = dot(oh_kj, sT_r[0])

    pd = d_col - d_row

    w0 = w0_r[...]
    c = c_r[...]
    h = jnp.maximum(pd[:, :, None] * w0[0][None, None, :] + c[0][None, None, :], 0.0)
    h = h.reshape(_K * _K, _H)
    for l in range(wh_r.shape[0]):
        h = jnp.maximum(dot(h, wh_r[l]) + bh_r[l][None, :], 0.0)
    o = jnp.sum(h.reshape(_K, _K, _H) * wo_r[...][0][None, None, :], axis=2)
    o = o + bo_r[0, 0]
    weight = jnp.maximum(o, 0.0) + jnp.log1p(jnp.exp(-jnp.abs(o)))

    t_dist = t_col - t_row
    s_dist = s_col - s_row
    target = (jnp.sign(t_dist) + 1.0) * 0.5
    bce = (jnp.maximum(s_dist, 0.0) - s_dist * target
           + jnp.log1p(jnp.exp(-jnp.abs(s_dist))))
    rk = jax.lax.broadcasted_iota(jnp.int32, (_K, _K), 0)
    ck = jax.lax.broadcasted_iota(jnp.int32, (_K, _K), 1)
    mask = (rk < ck).astype(jnp.float32)

    acc_r[0, 0] += jnp.sum(bce * weight * mask)
    acc_r[0, 1] += jnp.sqrt(jnp.sum(weight * weight))

    @pl.when(b == _B - 1)
    def _final():
        out_r[0, 0] = (acc_r[0, 0] / (_B * _NPAIR)
                       + wd_r[0, 0] * _B / acc_r[0, 1])


def kernel(gt, t_score, s_score, sample_dist, W_in, b_in, W_h, b_h, W_out,
           b_out, weight_decay):
    del gt
    sd = sample_dist.astype(jnp.float32)
    t = t_score.astype(jnp.float32)
    s = s_score.astype(jnp.float32)
    w0 = W_in[0].reshape(1, _H)
    c = (_UC * W_in[2] + b_in).reshape(1, _H)
    bh = b_h.astype(jnp.float32)
    wo = W_out[:, 0].reshape(1, _H)
    bo = jnp.asarray(b_out, jnp.float32).reshape(1, 1)
    wd = jnp.asarray(weight_decay, jnp.float32).reshape(1, 1)

    smem = functools.partial(pl.BlockSpec, memory_space=pltpu.SMEM)
    full = lambda *shape: pl.BlockSpec(shape, lambda b: (0,) * len(shape))
    out = pl.pallas_call(
        _loss_kernel,
        grid=(_B,),
        in_specs=[
            smem(),
            smem(),
            pl.BlockSpec((1, 1, _N), lambda b: (b, 0, 0)),
            pl.BlockSpec((1, _N, 1), lambda b: (b, 0, 0)),
            pl.BlockSpec((1, 1, _N), lambda b: (b, 0, 0)),
            pl.BlockSpec((1, _N, 1), lambda b: (b, 0, 0)),
            pl.BlockSpec((1, 1, _N), lambda b: (b, 0, 0)),
            pl.BlockSpec((1, _N, 1), lambda b: (b, 0, 0)),
            full(1, _H),
            full(1, _H),
            full(W_h.shape[0], _H, _H),
            full(W_h.shape[0], _H),
            full(1, _H),
        ],
        out_specs=pl.BlockSpec(memory_space=pltpu.SMEM),
        out_shape=jax.ShapeDtypeStruct((1, 1), jnp.float32),
        scratch_shapes=[pltpu.SMEM((1, 2), jnp.float32)],
        compiler_params=pltpu.CompilerParams(
            dimension_semantics=("arbitrary",)),
    )(wd, bo, sd[:, None, :], sd[:, :, None], t[:, None, :], t[:, :, None],
      s[:, None, :], s[:, :, None], w0, c, W_h.astype(jnp.float32), bh, wo)
    return out[0, 0]


# speedup vs baseline: 2.8937x; 2.8937x over previous
import functools

import numpy as np
import jax
import jax.numpy as jnp
from jax.experimental import pallas as pl
from jax.experimental.pallas import tpu as pltpu

_B, _N = 32, 512
_K = 128
_H = 64
_P = _K * _K
_NPAIR = _K * (_K - 1) // 2
_UC = float(np.sqrt(_N * (_N + 1) / 12.0))


def _build_consts():
    i = np.arange(_P) // _K
    j = np.arange(_P) % _K
    diff = np.zeros((_K, _P), np.float32)
    diff[i, np.arange(_P)] += 1.0
    diff[j, np.arange(_P)] -= 1.0
    mask = (i < j).astype(np.float32)[None, :]
    ones = np.ones((1, _P), np.float32)
    return diff, mask, ones


_DIFF_NP, _MASK_NP, _ONES_NP = _build_consts()


def _loss_kernel(wd_r, bo_r, sts_r, sdT_r, diff_r, mask_r, ones_r,
                 w1a_r, wha_r, wo_r, out_r, acc_r):
    b = pl.program_id(0)

    @pl.when(b == 0)
    def _init():
        acc_r[0, 0] = 0.0
        acc_r[0, 1] = 0.0

    dot = functools.partial(jnp.dot, preferred_element_type=jnp.float32)

    x_row = sts_r[0, 0:1]
    x_col = sdT_r[0]

    ri = jax.lax.broadcasted_iota(jnp.int32, (_N, _N), 0)
    cj = jax.lax.broadcasted_iota(jnp.int32, (_N, _N), 1)
    eq = x_row == x_col
    m_i = ((x_row > x_col) | (eq & (cj < ri))).astype(jnp.int32)
    rank_col = jnp.sum(m_i, axis=1, keepdims=True)

    kk_col = jax.lax.broadcasted_iota(jnp.int32, (_N, _K), 1)
    oh_ik = (rank_col == kk_col).astype(jnp.float32)

    sel = dot(sts_r[0], oh_ik)
    dts = dot(sel, diff_r[...])

    pd = dts[0:1]
    p2 = jnp.concatenate([pd, ones_r[...]], axis=0)
    h = jnp.maximum(dot(w1a_r[...], p2), 0.0)
    for l in range(wha_r.shape[0]):
        ha = jnp.concatenate([h, ones_r[...]], axis=0)
        h = jnp.maximum(dot(wha_r[l], ha), 0.0)
    o = dot(wo_r[...], h) + bo_r[0, 0]
    w = jnp.maximum(o, 0.0) + jnp.log1p(jnp.exp(-jnp.abs(o)))

    t_dist = dts[1:2]
    s_dist = dts[2:3]
    target = (jnp.sign(t_dist) + 1.0) * 0.5
    bce = (jnp.maximum(s_dist, 0.0) - s_dist * target
           + jnp.log1p(jnp.exp(-jnp.abs(s_dist))))

    acc_r[0, 0] += jnp.sum(bce * w * mask_r[...])
    acc_r[0, 1] += jnp.sqrt(jnp.sum(w * w))

    @pl.when(b == _B - 1)
    def _final():
        out_r[0, 0] = (acc_r[0, 0] / (_B * _NPAIR)
                       + wd_r[0, 0] * _B / acc_r[0, 1])


def kernel(gt, t_score, s_score, sample_dist, W_in, b_in, W_h, b_h, W_out,
           b_out, weight_decay):
    del gt
    sd = sample_dist.astype(jnp.float32)
    sts = jnp.stack([sd, t_score.astype(jnp.float32),
                     s_score.astype(jnp.float32)], axis=1)
    c = _UC * W_in[2] + b_in
    w1a = jnp.stack([W_in[0], c], axis=1)
    wha = jnp.concatenate(
        [jnp.swapaxes(W_h, 1, 2), b_h[:, :, None]], axis=2).astype(jnp.float32)
    wo = W_out[:, 0].reshape(1, _H)
    bo = jnp.asarray(b_out, jnp.float32).reshape(1, 1)
    wd = jnp.asarray(weight_decay, jnp.float32).reshape(1, 1)

    smem = functools.partial(pl.BlockSpec, memory_space=pltpu.SMEM)
    full = lambda *shape: pl.BlockSpec(shape, lambda b: (0,) * len(shape))
    out = pl.pallas_call(
        _loss_kernel,
        grid=(_B,),
        in_specs=[
            smem(),
            smem(),
            pl.BlockSpec((1, 3, _N), lambda b: (b, 0, 0)),
            pl.BlockSpec((1, _N, 1), lambda b: (b, 0, 0)),
            full(_K, _P),
            full(1, _P),
            full(1, _P),
            full(_H, 2),
            full(W_h.shape[0], _H, _H + 1),
            full(1, _H),
        ],
        out_specs=pl.BlockSpec(memory_space=pltpu.SMEM),
        out_shape=jax.ShapeDtypeStruct((1, 1), jnp.float32),
        scratch_shapes=[pltpu.SMEM((1, 2), jnp.float32)],
        compiler_params=pltpu.CompilerParams(
            dimension_semantics=("arbitrary",)),
    )(wd, bo, sts, sd[:, :, None], jnp.asarray(_DIFF_NP),
      jnp.asarray(_MASK_NP), jnp.asarray(_ONES_NP), w1a, wha, wo)
    return out[0, 0]


# speedup vs baseline: 2.9893x; 1.0330x over previous
import functools

import numpy as np
import jax
import jax.numpy as jnp
from jax.experimental import pallas as pl
from jax.experimental.pallas import tpu as pltpu

_B, _N = 32, 512
_K = 128
_H = 64
_P = _K * _K
_NPAIR = _K * (_K - 1) // 2
_UC = float(np.sqrt(_N * (_N + 1) / 12.0))


def _build_consts():
    i = np.arange(_P) // _K
    j = np.arange(_P) % _K
    diff = np.zeros((_K, _P), np.float32)
    diff[i, np.arange(_P)] += 1.0
    diff[j, np.arange(_P)] -= 1.0
    mask = (i < j).astype(np.float32)[None, :]
    return diff, mask


_DIFF_NP, _MASK_NP = _build_consts()


def _loss_kernel(wd_r, bo_r, sts_r, sdT_r, diff_r, mask_r,
                 w0_r, c_r, whb_r, bh_r, wo_r, out_r, acc_r):
    b = pl.program_id(0)

    @pl.when(b == 0)
    def _init():
        acc_r[0, 0] = 0.0
        acc_r[0, 1] = 0.0

    dot = functools.partial(jnp.dot, preferred_element_type=jnp.float32)

    x_row = sts_r[0, 0:1]
    x_col = sdT_r[0]

    ri = jax.lax.broadcasted_iota(jnp.int32, (_N, _N), 0)
    cj = jax.lax.broadcasted_iota(jnp.int32, (_N, _N), 1)
    eq = x_row == x_col
    m_i = ((x_row > x_col) | (eq & (cj < ri))).astype(jnp.int32)
    rank_col = jnp.sum(m_i, axis=1, keepdims=True)

    kk_col = jax.lax.broadcasted_iota(jnp.int32, (_N, _K), 1)
    oh_ik = (rank_col == kk_col).astype(jnp.float32)

    sel = dot(sts_r[0], oh_ik)
    lhs = jnp.concatenate([dot(w0_r[...], sel[0:1]), sel[1:3]], axis=0)
    big = dot(lhs, diff_r[...])

    h = jnp.maximum(big[0:_H] + c_r[...], 0.0)
    for l in range(whb_r.shape[0]):
        h = jnp.maximum(dot(whb_r[l], h) + bh_r[l], 0.0)
    o = dot(wo_r[...], h) + bo_r[0, 0]
    w = jnp.maximum(o, 0.0) + jnp.log1p(jnp.exp(-jnp.abs(o)))

    t_dist = big[_H:_H + 1]
    s_dist = big[_H + 1:_H + 2]
    target = (jnp.sign(t_dist) + 1.0) * 0.5
    bce = (jnp.maximum(s_dist, 0.0) - s_dist * target
           + jnp.log1p(jnp.exp(-jnp.abs(s_dist))))

    acc_r[0, 0] += jnp.sum(bce * w * mask_r[...])
    acc_r[0, 1] += jnp.sqrt(jnp.sum(w * w))

    @pl.when(b == _B - 1)
    def _final():
        out_r[0, 0] = (acc_r[0, 0] / (_B * _NPAIR)
                       + wd_r[0, 0] * _B / acc_r[0, 1])


def kernel(gt, t_score, s_score, sample_dist, W_in, b_in, W_h, b_h, W_out,
           b_out, weight_decay):
    del gt
    sd = sample_dist.astype(jnp.float32)
    sts = jnp.stack([sd, t_score.astype(jnp.float32),
                     s_score.astype(jnp.float32)], axis=1)
    w0 = W_in[0].reshape(_H, 1)
    c = (_UC * W_in[2] + b_in).reshape(_H, 1)
    whb = jnp.swapaxes(W_h, 1, 2).astype(jnp.float32)
    bh = b_h.astype(jnp.float32)[:, :, None]
    wo = W_out[:, 0].reshape(1, _H)
    bo = jnp.asarray(b_out, jnp.float32).reshape(1, 1)
    wd = jnp.asarray(weight_decay, jnp.float32).reshape(1, 1)

    smem = functools.partial(pl.BlockSpec, memory_space=pltpu.SMEM)
    full = lambda *shape: pl.BlockSpec(shape, lambda b: (0,) * len(shape))
    out = pl.pallas_call(
        _loss_kernel,
        grid=(_B,),
        in_specs=[
            smem(),
            smem(),
            pl.BlockSpec((1, 3, _N), lambda b: (b, 0, 0)),
            pl.BlockSpec((1, _N, 1), lambda b: (b, 0, 0)),
            full(_K, _P),
            full(1, _P),
            full(_H, 1),
            full(_H, 1),
            full(W_h.shape[0], _H, _H),
            full(W_h.shape[0], _H, 1),
            full(1, _H),
        ],
        out_specs=pl.BlockSpec(memory_space=pltpu.SMEM),
        out_shape=jax.ShapeDtypeStruct((1, 1), jnp.float32),
        scratch_shapes=[pltpu.SMEM((1, 2), jnp.float32)],
        compiler_params=pltpu.CompilerParams(
            dimension_semantics=("arbitrary",)),
    )(wd, bo, sts, sd[:, :, None], jnp.asarray(_DIFF_NP),
      jnp.asarray(_MASK_NP), w0, c, whb, bh, wo)
    return out[0, 0]


# speedup vs baseline: 3.2028x; 1.0714x over previous
import functools

import numpy as np
import jax
import jax.numpy as jnp
from jax.experimental import pallas as pl
from jax.experimental.pallas import tpu as pltpu

_B, _N = 32, 512
_K = 128
_H = 64
_P = _K * _K
_NPAIR = _K * (_K - 1) // 2
_UC = float(np.sqrt(_N * (_N + 1) / 12.0))


def _build_consts():
    i = np.arange(_P) // _K
    j = np.arange(_P) % _K
    diff = np.zeros((_K, _P), np.float32)
    diff[i, np.arange(_P)] += 1.0
    diff[j, np.arange(_P)] -= 1.0
    mask = (i < j).astype(np.float32)[None, :]
    return diff, mask


_DIFF_NP, _MASK_NP = _build_consts()


_RPB = 2


def _loss_kernel(wd_r, bo_r, sts_r, sdT_r, diff_r, mask_r,
                 w0_r, c_r, whb_r, bh_r, wo_r, out_r, acc_r):
    b = pl.program_id(0)

    @pl.when(b == 0)
    def _init():
        acc_r[0, 0] = 0.0
        acc_r[0, 1] = 0.0

    dot = functools.partial(jnp.dot, preferred_element_type=jnp.float32)

    bsum, nsum = 0.0, 0.0
    for r in range(_RPB):
        x_row = sts_r[r, 0:1]
        x_col = sdT_r[r]

        ri = jax.lax.broadcasted_iota(jnp.int32, (_N, _N), 0)
        cj = jax.lax.broadcasted_iota(jnp.int32, (_N, _N), 1)
        eq = x_row == x_col
        m_i = ((x_row > x_col) | (eq & (cj < ri))).astype(jnp.int32)
        rank_col = jnp.sum(m_i, axis=1, keepdims=True)

        kk_col = jax.lax.broadcasted_iota(jnp.int32, (_N, _K), 1)
        oh_ik = (rank_col == kk_col).astype(jnp.float32)

        sel = dot(sts_r[r], oh_ik)
        lhs = jnp.concatenate([dot(w0_r[...], sel[0:1]), sel[1:3]], axis=0)
        big = dot(lhs, diff_r[...])

        h = jnp.maximum(big[0:_H] + c_r[...], 0.0)
        for l in range(whb_r.shape[0]):
            h = jnp.maximum(dot(whb_r[l], h) + bh_r[l], 0.0)
        o = dot(wo_r[...], h) + bo_r[0, 0]
        w = jnp.maximum(o, 0.0) + jnp.log1p(jnp.exp(-jnp.abs(o)))

        t_dist = big[_H:_H + 1]
        s_dist = big[_H + 1:_H + 2]
        target = (jnp.sign(t_dist) + 1.0) * 0.5
        bce = (jnp.maximum(s_dist, 0.0) - s_dist * target
               + jnp.log1p(jnp.exp(-jnp.abs(s_dist))))

        bsum += jnp.sum(bce * w * mask_r[...])
        nsum += jnp.sqrt(jnp.sum(w * w))

    acc_r[0, 0] += bsum
    acc_r[0, 1] += nsum

    @pl.when(b == _B // _RPB - 1)
    def _final():
        out_r[0, 0] = (acc_r[0, 0] / (_B * _NPAIR)
                       + wd_r[0, 0] * _B / acc_r[0, 1])


def kernel(gt, t_score, s_score, sample_dist, W_in, b_in, W_h, b_h, W_out,
           b_out, weight_decay):
    del gt
    sd = sample_dist.astype(jnp.float32)
    sts = jnp.stack([sd, t_score.astype(jnp.float32),
                     s_score.astype(jnp.float32)], axis=1)
    w0 = W_in[0].reshape(_H, 1)
    c = (_UC * W_in[2] + b_in).reshape(_H, 1)
    whb = jnp.swapaxes(W_h, 1, 2).astype(jnp.float32)
    bh = b_h.astype(jnp.float32)[:, :, None]
    wo = W_out[:, 0].reshape(1, _H)
    bo = jnp.asarray(b_out, jnp.float32).reshape(1, 1)
    wd = jnp.asarray(weight_decay, jnp.float32).reshape(1, 1)

    smem = functools.partial(pl.BlockSpec, memory_space=pltpu.SMEM)
    full = lambda *shape: pl.BlockSpec(shape, lambda b: (0,) * len(shape))
    out = pl.pallas_call(
        _loss_kernel,
        grid=(_B // _RPB,),
        in_specs=[
            smem(),
            smem(),
            pl.BlockSpec((_RPB, 3, _N), lambda b: (b, 0, 0)),
            pl.BlockSpec((_RPB, _N, 1), lambda b: (b, 0, 0)),
            full(_K, _P),
            full(1, _P),
            full(_H, 1),
            full(_H, 1),
            full(W_h.shape[0], _H, _H),
            full(W_h.shape[0], _H, 1),
            full(1, _H),
        ],
        out_specs=pl.BlockSpec(memory_space=pltpu.SMEM),
        out_shape=jax.ShapeDtypeStruct((1, 1), jnp.float32),
        scratch_shapes=[pltpu.SMEM((1, 2), jnp.float32)],
        compiler_params=pltpu.CompilerParams(
            dimension_semantics=("arbitrary",)),
    )(wd, bo, sts, sd[:, :, None], jnp.asarray(_DIFF_NP),
      jnp.asarray(_MASK_NP), w0, c, whb, bh, wo)
    return out[0, 0]
